# baseline (speedup 1.0000x reference)
import functools

import jax
import jax.numpy as jnp
from jax import lax
from jax.experimental import pallas as pl
from jax.experimental.pallas import tpu as pltpu
from jax.experimental.pallas import tpu_sc as plsc

_B, _S, _D = 4, 8192, 1024
_L = 16
_W = 128
_DW = _D // _W
_NC, _NS = 2, 16
_NW = _NC * _NS
_SEQ_W = _S // _NW
_C = 8
_NCH = _SEQ_W // _C
_R = _C * _DW
_NSLOT = 3

_mesh = plsc.VectorSubcoreMesh(core_axis_name="c", subcore_axis_name="s")


@functools.partial(
    pl.kernel,
    mesh=_mesh,
    out_type=jax.ShapeDtypeStruct((_B, _S * _DW, _W), jnp.float32),
    scratch_types=[
        pltpu.VMEM((_NSLOT, _R, _W), jnp.float32),
        pltpu.VMEM((_NSLOT, _B, _R, _W), jnp.float32),
        pltpu.SemaphoreType.DMA,
        pltpu.SemaphoreType.DMA,
        pltpu.SemaphoreType.DMA,
    ],
)
def _sc_add(x_hbm, emb_hbm, out_hbm, ebuf, xbuf, esem, xsem, osem):
    wid = lax.axis_index("s") * _NC + lax.axis_index("c")
    seq0 = wid * _SEQ_W

    def e_off(k):
        return (seq0 + k * _C) * _DW

    def start_in(k):
        e0 = e_off(k)
        s = lax.rem(k, _NSLOT)
        pltpu.make_async_copy(
            emb_hbm.at[pl.ds(e0, _R)], ebuf.at[lax.rem(k, _NSLOT)], esem
        ).start()
        pltpu.make_async_copy(
            x_hbm.at[:, pl.ds(e0, _R)], xbuf.at[s], xsem
        ).start()

    def wait_in(k):
        s = lax.rem(k, _NSLOT)
        pltpu.make_async_copy(
            emb_hbm.at[pl.ds(e_off(k), _R)], ebuf.at[lax.rem(k, _NSLOT)], esem
        ).wait()
        pltpu.make_async_copy(
            x_hbm.at[:, pl.ds(e_off(k), _R)], xbuf.at[s], xsem
        ).wait()

    def start_out(k):
        e0 = e_off(k)
        s = lax.rem(k, _NSLOT)
        pltpu.make_async_copy(
            xbuf.at[s], out_hbm.at[:, pl.ds(e0, _R)], osem
        ).start()

    def wait_out_chunk():
        pltpu.make_async_copy(
            xbuf.at[0], out_hbm.at[:, pl.ds(0, _R)], osem
        ).wait()

    def compute(k):
        s = lax.rem(k, _NSLOT)
        es = lax.rem(k, _NSLOT)

        @plsc.parallel_loop(0, _R, 1, unroll=4)
        def _(i):
            for g in range(_W // _L):
                sl = pl.ds(g * _L, _L)
                e = ebuf[es, i, sl]
                for b in range(_B):
                    xbuf[s, b, i, sl] = xbuf[s, b, i, sl] + e

    start_in(0)
    start_in(1)

    def body(k, carry):
        wait_in(k)
        compute(k)
        start_out(k)

        @pl.when(k >= 1)
        def _():
            wait_out_chunk()

        @pl.when(k + 2 < _NCH)
        def _():
            start_in(k + 2)

        return carry

    lax.fori_loop(0, _NCH, body, 0)
    wait_out_chunk()


def kernel(x, emb_weight):
    b, s, d = x.shape
    x2 = x.reshape(b, s * (d // _W), _W)
    e2 = emb_weight.reshape(s * (d // _W), _W)
    out = _sc_add(x2, e2)
    return out.reshape(b, s, d)


# speedup vs baseline: 4.5531x; 4.5531x over previous
import jax
import jax.numpy as jnp
from jax.experimental import pallas as pl

_BS = 2048


def _add_body(x_ref, emb_ref, out_ref):
    out_ref[...] = x_ref[...] + emb_ref[...]


def kernel(x, emb_weight):
    B, S, D = x.shape
    bs = min(_BS, S)
    grid = (S // bs, B)
    return pl.pallas_call(
        _add_body,
        grid=grid,
        in_specs=[
            pl.BlockSpec((1, bs, D), lambda s, b: (b, s, 0)),
            pl.BlockSpec((bs, D), lambda s, b: (s, 0)),
        ],
        out_specs=pl.BlockSpec((1, bs, D), lambda s, b: (b, s, 0)),
        out_shape=jax.ShapeDtypeStruct((B, S, D), x.dtype),
    )(x, emb_weight)
